# Initial kernel scaffold; baseline (speedup 1.0000x reference)
import functools

import jax
import jax.numpy as jnp
from jax import lax
from jax.experimental import pallas as pl
from jax.experimental.pallas import tpu as pltpu
from jax.experimental.pallas import tpu_sc as plsc

N_NODES = 10000
N_EDGES = 320000
NC = 2
NS = 16
CHUNK = 80
ROWS_PER_SUB = N_NODES // NS

_MESH = plsc.VectorSubcoreMesh(core_axis_name="c", subcore_axis_name="s")


def _degree_kernel(dst_hbm, degp_out, ones_v, dst_v, zero_v, acc1d):
  c = lax.axis_index("c")
  s = lax.axis_index("s")
  for k in range(CHUNK // 16):
    ones_v[pl.ds(k * 16, 16)] = jnp.full((16,), 1.0, jnp.float32)
  @pl.when(s == 0)
  def _():
    def zloop(i, _):
      zero_v[pl.ds(i * 16, 16)] = jnp.zeros((16,), jnp.float32)
      return 0
    lax.fori_loop(0, N_NODES // 16, zloop, 0)
    pltpu.sync_copy(zero_v, acc1d)
  plsc.subcore_barrier()
  epw = N_EDGES // (NC * NS)
  base = c * (N_EDGES // NC) + s * epw
  def body(j, _):
    pltpu.sync_copy(dst_hbm.at[pl.ds(base + j * CHUNK, CHUNK)], dst_v)
    pltpu.sync_copy(ones_v, acc1d.at[dst_v], add=True)
    return 0
  lax.fori_loop(0, epw // CHUNK, body, 0)
  plsc.subcore_barrier()
  @pl.when(s == 0)
  def _():
    pltpu.sync_copy(acc1d, degp_out.at[c])


_degree = pl.kernel(
    _degree_kernel,
    out_type=jax.ShapeDtypeStruct((NC, N_NODES), jnp.float32),
    mesh=_MESH,
    scratch_types=[
        pltpu.VMEM((CHUNK,), jnp.float32),
        pltpu.VMEM((CHUNK,), jnp.int32),
        pltpu.VMEM((N_NODES,), jnp.float32),
        pltpu.MemorySpace.VMEM_SHARED((N_NODES,), jnp.float32),
    ],
)


def _propagate_kernel(h_hbm, src_hbm, dst_hbm, acc_out,
                      src_v, dst_v, rows_v, acc, sem,
                      *, feature_split):
  c = lax.axis_index("c")
  s = lax.axis_index("s")
  r0 = s * ROWS_PER_SUB
  if feature_split:
    pltpu.sync_copy(h_hbm.at[pl.ds(c * N_NODES + r0, ROWS_PER_SUB)],
                    acc.at[pl.ds(r0, ROWS_PER_SUB)])
    epw = N_EDGES // NS
    base = s * epw
  else:
    pltpu.sync_copy(h_hbm.at[pl.ds(r0, ROWS_PER_SUB)],
                    acc.at[pl.ds(r0, ROWS_PER_SUB)])
    epw = N_EDGES // (NC * NS)
    base = c * (N_EDGES // NC) + s * epw
  plsc.subcore_barrier()

  def body(j, _):
    off = base + j * CHUNK
    pltpu.sync_copy(src_hbm.at[pl.ds(off, CHUNK)], src_v)
    pltpu.sync_copy(dst_hbm.at[pl.ds(off, CHUNK)], dst_v)
    if feature_split:
      for k in range(CHUNK // 16):
        sl = pl.ds(k * 16, 16)
        src_v[sl] = src_v[sl] + c * N_NODES
    pltpu.async_copy(h_hbm.at[src_v], rows_v, sem).wait()
    pltpu.sync_copy(rows_v, acc.at[dst_v], add=True)
    return 0
  lax.fori_loop(0, epw // CHUNK, body, 0)
  plsc.subcore_barrier()
  pltpu.sync_copy(acc.at[pl.ds(r0, ROWS_PER_SUB)],
                  acc_out.at[c, pl.ds(r0, ROWS_PER_SUB)])


def _make_propagate(feature_split):
  return pl.kernel(
      functools.partial(_propagate_kernel, feature_split=feature_split),
      out_type=jax.ShapeDtypeStruct((NC, N_NODES, 128), jnp.float32),
      mesh=_MESH,
      scratch_types=[
          pltpu.VMEM((CHUNK,), jnp.int32),
          pltpu.VMEM((CHUNK,), jnp.int32),
          pltpu.VMEM((CHUNK, 128), jnp.float32),
          pltpu.MemorySpace.VMEM_SHARED((N_NODES, 128), jnp.float32),
          pltpu.SemaphoreType.DMA,
      ],
      name="propagate_fs" if feature_split else "propagate_es",
  )


_propagate_l1 = _make_propagate(True)
_propagate_l2 = _make_propagate(False)


_BLK = 512
_GRID = (N_NODES + _BLK - 1) // _BLK


def _dinv_from(degp):
  deg = jnp.sum(degp, axis=0) + 1.0
  return lax.rsqrt(deg)


def _tc1_kernel(x_ref, w1_ref, degp_ref, a_ref, h1s_ref):
  del a_ref
  dinv = _dinv_from(degp_ref[...])
  h = jnp.dot(x_ref[...], w1_ref[...], preferred_element_type=jnp.float32)
  hs = h * dinv[:, None]
  h1s_ref[0] = hs[:, :128]
  h1s_ref[1] = hs[:, 128:]


def _tc2_kernel(agg1_ref, degp_ref, b1_ref, w2_ref, a_ref, h2s_ref):
  a = a_ref[0, 0]
  dinv = _dinv_from(degp_ref[...])
  h1 = jnp.concatenate([agg1_ref[0], agg1_ref[1]], axis=-1)
  h1 = h1 * dinv[:, None] + b1_ref[...][None, :]
  h1 = jnp.where(h1 >= 0, h1, a * h1)
  h2 = jnp.dot(h1, w2_ref[...], preferred_element_type=jnp.float32)
  h2s_ref[...] = h2 * dinv[:, None]


def _tc3_kernel(acc2_ref, h2s_ref, degp_ref, b2_ref, a_ref, out_ref):
  a = a_ref[0, 0]
  dinv = _dinv_from(degp_ref[...])
  agg = acc2_ref[0] + acc2_ref[1] - h2s_ref[...]
  out = agg * dinv[:, None] + b2_ref[...][None, :]
  out_ref[...] = jnp.where(out >= 0, out, a * out)


def _row_spec(shape_tail):
  return pl.BlockSpec((_BLK,) + shape_tail, lambda i: (i,) + (0,) * len(shape_tail))


_degp_spec = pl.BlockSpec((NC, _BLK), lambda i: (0, i))
_smem_spec = pl.BlockSpec(memory_space=pltpu.MemorySpace.SMEM)


def _full_spec(ndim):
  return pl.BlockSpec(None, lambda i: (0,) * ndim)


_tc1 = pl.pallas_call(
    _tc1_kernel,
    grid=(_GRID,),
    in_specs=[_row_spec((128,)), _full_spec(2), _degp_spec, _smem_spec],
    out_specs=pl.BlockSpec((NC, _BLK, 128), lambda i: (0, i, 0)),
    out_shape=jax.ShapeDtypeStruct((NC, N_NODES, 128), jnp.float32),
)

_tc2 = pl.pallas_call(
    _tc2_kernel,
    grid=(_GRID,),
    in_specs=[pl.BlockSpec((NC, _BLK, 128), lambda i: (0, i, 0)),
              _degp_spec, _full_spec(1), _full_spec(2), _smem_spec],
    out_specs=_row_spec((128,)),
    out_shape=jax.ShapeDtypeStruct((N_NODES, 128), jnp.float32),
)

_tc3 = pl.pallas_call(
    _tc3_kernel,
    grid=(_GRID,),
    in_specs=[pl.BlockSpec((NC, _BLK, 128), lambda i: (0, i, 0)),
              _row_spec((128,)), _degp_spec, _full_spec(1), _smem_spec],
    out_specs=_row_spec((128,)),
    out_shape=jax.ShapeDtypeStruct((N_NODES, 128), jnp.float32),
)


@jax.jit
def kernel(x, edge_index, W1, b1, W2, b2, prelu_a):
  src = edge_index[0].astype(jnp.int32)
  dst = edge_index[1].astype(jnp.int32)
  a = jnp.reshape(prelu_a.astype(jnp.float32), (1, 1))

  degp = _degree(dst)
  h1s = _tc1(x, W1, degp, a)
  agg1 = _propagate_l1(h1s.reshape(NC * N_NODES, 128), src, dst)
  h2s = _tc2(agg1, degp, b1, W2, a)
  acc2 = _propagate_l2(h2s, src, dst)
  return _tc3(acc2, h2s, degp, b2, a)


# speedup vs baseline: 10.4879x; 10.4879x over previous
import functools

import jax
import jax.numpy as jnp
from jax import lax
from jax.experimental import pallas as pl
from jax.experimental.pallas import tpu as pltpu
from jax.experimental.pallas import tpu_sc as plsc

N_NODES = 10000
N_EDGES = 320000
NC = 2
NS = 16
CHUNK = 80
ROWS_A = 632
ROWS_LAST = N_NODES - (NS - 1) * ROWS_A


def _degree_kernel(dst_hbm, degp_out, ones_v, dst_v, zero_v, acc1d):
  c = lax.axis_index("c")
  s = lax.axis_index("s")
  for k in range(CHUNK // 16):
    ones_v[pl.ds(k * 16, 16)] = jnp.full((16,), 1.0, jnp.float32)
  @pl.when(s == 0)
  def _():
    def zloop(i, _):
      zero_v[pl.ds(i * 16, 16)] = jnp.zeros((16,), jnp.float32)
      return 0
    lax.fori_loop(0, N_NODES // 16, zloop, 0)
    pltpu.sync_copy(zero_v, acc1d)
  plsc.subcore_barrier()
  epw = N_EDGES // (NC * NS)
  base = c * (N_EDGES // NC) + s * epw
  def body(j, _):
    pltpu.sync_copy(dst_hbm.at[pl.ds(base + j * CHUNK, CHUNK)], dst_v)
    pltpu.sync_copy(ones_v, acc1d.at[dst_v], add=True)
    return 0
  lax.fori_loop(0, epw // CHUNK, body, 0)
  plsc.subcore_barrier()
  @pl.when(s == 0)
  def _():
    pltpu.sync_copy(acc1d, zero_v)
    pltpu.sync_copy(zero_v, degp_out.at[pl.ds(c * N_NODES, N_NODES)])


@functools.cache
def _mesh():
  return plsc.VectorSubcoreMesh(core_axis_name="c", subcore_axis_name="s",
                                num_cores=NC, num_subcores=NS)


@functools.cache
def _degree():
  return pl.kernel(
      _degree_kernel,
      out_type=jax.ShapeDtypeStruct((NC * N_NODES,), jnp.float32),
      mesh=_mesh(),
      scratch_types=[
          pltpu.VMEM((CHUNK,), jnp.float32),
          pltpu.VMEM((CHUNK,), jnp.int32),
          pltpu.VMEM((N_NODES,), jnp.float32),
          pltpu.MemorySpace.VMEM_SHARED((N_NODES,), jnp.float32),
      ],
      name="degree",
  )


def _propagate_kernel(h_hbm, src_hbm, dst_hbm, acc_out,
                      src_v, dst_v, rows_v, acc, sem,
                      *, feature_split):
  c = lax.axis_index("c")
  s = lax.axis_index("s")
  r0 = s * ROWS_A
  tbl0 = c * N_NODES if feature_split else 0
  @pl.when(s < NS - 1)
  def _():
    pltpu.sync_copy(h_hbm.at[pl.ds(tbl0 + r0, ROWS_A)],
                    acc.at[pl.ds(r0, ROWS_A)])
  @pl.when(s == NS - 1)
  def _():
    pltpu.sync_copy(h_hbm.at[pl.ds(tbl0 + r0, ROWS_LAST)],
                    acc.at[pl.ds(r0, ROWS_LAST)])
  if feature_split:
    epw = N_EDGES // NS
    base = s * epw
  else:
    epw = N_EDGES // (NC * NS)
    base = c * (N_EDGES // NC) + s * epw
  plsc.subcore_barrier()

  def body(j, _):
    off = base + j * CHUNK
    pltpu.sync_copy(src_hbm.at[pl.ds(off, CHUNK)], src_v)
    pltpu.sync_copy(dst_hbm.at[pl.ds(off, CHUNK)], dst_v)
    if feature_split:
      for k in range(CHUNK // 16):
        sl = pl.ds(k * 16, 16)
        src_v[sl] = src_v[sl] + c * N_NODES
    pltpu.async_copy(h_hbm.at[src_v], rows_v, sem).wait()
    pltpu.sync_copy(rows_v, acc.at[dst_v], add=True)
    return 0
  lax.fori_loop(0, epw // CHUNK, body, 0)
  plsc.subcore_barrier()
  @pl.when(s < NS - 1)
  def _():
    pltpu.sync_copy(acc.at[pl.ds(r0, ROWS_A)],
                    acc_out.at[c, pl.ds(r0, ROWS_A)])
  @pl.when(s == NS - 1)
  def _():
    pltpu.sync_copy(acc.at[pl.ds(r0, ROWS_LAST)],
                    acc_out.at[c, pl.ds(r0, ROWS_LAST)])


@functools.cache
def _make_propagate(feature_split):
  return pl.kernel(
      functools.partial(_propagate_kernel, feature_split=feature_split),
      out_type=jax.ShapeDtypeStruct((NC, N_NODES, 128), jnp.float32),
      mesh=_mesh(),
      scratch_types=[
          pltpu.VMEM((CHUNK,), jnp.int32),
          pltpu.VMEM((CHUNK,), jnp.int32),
          pltpu.VMEM((CHUNK, 128), jnp.float32),
          pltpu.MemorySpace.VMEM_SHARED((N_NODES, 128), jnp.float32),
          pltpu.SemaphoreType.DMA,
      ],
      name="propagate_fs" if feature_split else "propagate_es",
  )


_BLK = 512
_GRID = (N_NODES + _BLK - 1) // _BLK


def _dinv_from(degp):
  deg = jnp.sum(degp, axis=0) + 1.0
  return lax.rsqrt(deg)


def _tc1_kernel(x_ref, w1_ref, degp_ref, a_ref, h1s_ref):
  del a_ref
  dinv = _dinv_from(degp_ref[...])
  h = jnp.dot(x_ref[...], w1_ref[...], preferred_element_type=jnp.float32)
  hs = h * dinv[:, None]
  h1s_ref[0] = hs[:, :128]
  h1s_ref[1] = hs[:, 128:]


def _tc2_kernel(agg1_ref, degp_ref, b1_ref, w2_ref, a_ref, h2s_ref):
  a = a_ref[0, 0]
  dinv = _dinv_from(degp_ref[...])
  h1 = jnp.concatenate([agg1_ref[0], agg1_ref[1]], axis=-1)
  h1 = h1 * dinv[:, None] + b1_ref[...][None, :]
  h1 = jnp.where(h1 >= 0, h1, a * h1)
  h2 = jnp.dot(h1, w2_ref[...], preferred_element_type=jnp.float32)
  h2s_ref[...] = h2 * dinv[:, None]


def _tc3_kernel(acc2_ref, h2s_ref, degp_ref, b2_ref, a_ref, out_ref):
  a = a_ref[0, 0]
  dinv = _dinv_from(degp_ref[...])
  agg = acc2_ref[0] + acc2_ref[1] - h2s_ref[...]
  out = agg * dinv[:, None] + b2_ref[...][None, :]
  out_ref[...] = jnp.where(out >= 0, out, a * out)


def _row_spec(shape_tail):
  return pl.BlockSpec((_BLK,) + shape_tail, lambda i: (i,) + (0,) * len(shape_tail))


_degp_spec = pl.BlockSpec((NC, _BLK), lambda i: (0, i))
_smem_spec = pl.BlockSpec(memory_space=pltpu.MemorySpace.SMEM)


def _full_spec(ndim):
  return pl.BlockSpec(None, lambda i: (0,) * ndim)


_tc1 = pl.pallas_call(
    _tc1_kernel,
    grid=(_GRID,),
    in_specs=[_row_spec((128,)), _full_spec(2), _degp_spec, _smem_spec],
    out_specs=pl.BlockSpec((NC, _BLK, 128), lambda i: (0, i, 0)),
    out_shape=jax.ShapeDtypeStruct((NC, N_NODES, 128), jnp.float32),
)

_tc2 = pl.pallas_call(
    _tc2_kernel,
    grid=(_GRID,),
    in_specs=[pl.BlockSpec((NC, _BLK, 128), lambda i: (0, i, 0)),
              _degp_spec, _full_spec(1), _full_spec(2), _smem_spec],
    out_specs=_row_spec((128,)),
    out_shape=jax.ShapeDtypeStruct((N_NODES, 128), jnp.float32),
)

_tc3 = pl.pallas_call(
    _tc3_kernel,
    grid=(_GRID,),
    in_specs=[pl.BlockSpec((NC, _BLK, 128), lambda i: (0, i, 0)),
              _row_spec((128,)), _degp_spec, _full_spec(1), _smem_spec],
    out_specs=_row_spec((128,)),
    out_shape=jax.ShapeDtypeStruct((N_NODES, 128), jnp.float32),
)


@jax.jit
def kernel(x, edge_index, W1, b1, W2, b2, prelu_a):
  src = edge_index[0].astype(jnp.int32)
  dst = edge_index[1].astype(jnp.int32)
  a = jnp.reshape(prelu_a.astype(jnp.float32), (1, 1))

  degp = _degree()(dst).reshape(NC, N_NODES)
  h1s = _tc1(x, W1, degp, a)
  agg1 = _make_propagate(True)(h1s.reshape(NC * N_NODES, 128), src, dst)
  h2s = _tc2(agg1, degp, b1, W2, a)
  acc2 = _make_propagate(False)(h2s, src, dst)
  return _tc3(acc2, h2s, degp, b2, a)
